# Initial kernel scaffold; baseline (speedup 1.0000x reference)
import jax
import jax.numpy as jnp
from jax.experimental import pallas as pl


def kernel(x, edge_weight, W0, b0, W1, b1, W2, b2, edge_index):
    raise NotImplementedError("write your pallas kernel here")


# speedup vs baseline: 3.7262x; 3.7262x over previous
import functools

import jax
import jax.numpy as jnp
from jax import lax
from jax.experimental import pallas as pl
from jax.experimental.pallas import tpu as pltpu
from jax.experimental.pallas import tpu_sc as plsc

N = 10000
E = 320000
D = 128
LANES = 16
NC = 2
NS = 16
EPT = E // (NC * NS)
C = 80
NCHUNK = EPT // C
STRIPE = 632
LAST_STRIPE = N - (NS - 1) * STRIPE


def _spmm_body(x_hbm, row_hbm, col_hbm, w_hbm, zeros_hbm, out_hbm,
               colbuf, rowbuf, wbuf, rows, acc, sem):
    c = lax.axis_index("c")
    s = lax.axis_index("s")

    @pl.when(s < NS - 1)
    def _():
        pltpu.sync_copy(zeros_hbm.at[pl.ds(s * STRIPE, STRIPE)],
                        acc.at[pl.ds(s * STRIPE, STRIPE)])

    @pl.when(s == NS - 1)
    def _():
        pltpu.sync_copy(zeros_hbm.at[pl.ds((NS - 1) * STRIPE, LAST_STRIPE)],
                        acc.at[pl.ds((NS - 1) * STRIPE, LAST_STRIPE)])

    plsc.subcore_barrier()

    tile_base = (c * NS + s) * EPT

    def chunk_body(k, carry):
        base = tile_base + k * C
        pltpu.sync_copy(col_hbm.at[pl.ds(base, C)], colbuf)
        pltpu.sync_copy(row_hbm.at[pl.ds(base, C)], rowbuf)
        pltpu.sync_copy(w_hbm.at[pl.ds(base, C)], wbuf)
        pltpu.async_copy(x_hbm.at[colbuf], rows, sem).wait()

        def group_body(g, carry2):
            w16 = wbuf[pl.ds(g * LANES, LANES)]
            for i in range(LANES):
                ws = w16.at[jnp.full((LANES,), i, jnp.int32)].get(
                    mode="promise_in_bounds")
                e = g * LANES + i
                for j in range(D // LANES):
                    sl = pl.ds(j * LANES, LANES)
                    rows[e, sl] = rows[e, sl] * ws
            return carry2

        lax.fori_loop(0, C // LANES, group_body, 0)
        pltpu.sync_copy(rows, acc.at[rowbuf], add=True)
        return carry

    lax.fori_loop(0, NCHUNK, chunk_body, 0)
    plsc.subcore_barrier()

    @pl.when(s < NS - 1)
    def _():
        pltpu.sync_copy(acc.at[pl.ds(s * STRIPE, STRIPE)],
                        out_hbm.at[c, pl.ds(s * STRIPE, STRIPE)])

    @pl.when(s == NS - 1)
    def _():
        pltpu.sync_copy(acc.at[pl.ds((NS - 1) * STRIPE, LAST_STRIPE)],
                        out_hbm.at[c, pl.ds((NS - 1) * STRIPE, LAST_STRIPE)])


@functools.partial(
    pl.kernel,
    mesh=plsc.VectorSubcoreMesh(core_axis_name="c", subcore_axis_name="s"),
    out_type=jax.ShapeDtypeStruct((NC, N, D), jnp.float32),
    scratch_types=[
        pltpu.VMEM((C,), jnp.int32),
        pltpu.VMEM((C,), jnp.int32),
        pltpu.VMEM((C,), jnp.float32),
        pltpu.VMEM((C, D), jnp.float32),
        pltpu.VMEM_SHARED((N, D), jnp.float32),
        pltpu.SemaphoreType.DMA,
    ],
)
def _spmm_sc(x_hbm, row_hbm, col_hbm, w_hbm, zeros_hbm, out_hbm,
             colbuf, rowbuf, wbuf, rows, acc, sem):
    _spmm_body(x_hbm, row_hbm, col_hbm, w_hbm, zeros_hbm, out_hbm,
               colbuf, rowbuf, wbuf, rows, acc, sem)


def _add_body(a_ref, b_ref, o_ref):
    o_ref[...] = a_ref[...] + b_ref[...]


def _pair_add(a, b):
    blk = 1000
    return pl.pallas_call(
        _add_body,
        grid=(N // blk,),
        in_specs=[pl.BlockSpec((blk, D), lambda i: (i, 0)),
                  pl.BlockSpec((blk, D), lambda i: (i, 0))],
        out_specs=pl.BlockSpec((blk, D), lambda i: (i, 0)),
        out_shape=jax.ShapeDtypeStruct((N, D), jnp.float32),
    )(a, b)


def _layer_body(relu, x0_ref, x1_ref, q0_ref, q1_ref, wc_ref, b_ref, o_ref):
    q = q0_ref[...] + q1_ref[...]
    cat = jnp.concatenate([x0_ref[...], x1_ref[...], q], axis=1)
    h = jnp.dot(cat, wc_ref[...], preferred_element_type=jnp.float32)
    h = h + b_ref[...]
    if relu:
        h = jnp.maximum(h, 0.0)
    o_ref[...] = h


def _layer_tc(x0, x1, q0, q1, wc, b2d, relu):
    blk = 1000
    return pl.pallas_call(
        functools.partial(_layer_body, relu),
        grid=(N // blk,),
        in_specs=[pl.BlockSpec((blk, D), lambda i: (i, 0)),
                  pl.BlockSpec((blk, D), lambda i: (i, 0)),
                  pl.BlockSpec((blk, D), lambda i: (i, 0)),
                  pl.BlockSpec((blk, D), lambda i: (i, 0)),
                  pl.BlockSpec((3 * D, D), lambda i: (0, 0)),
                  pl.BlockSpec((1, D), lambda i: (0, 0))],
        out_specs=pl.BlockSpec((blk, D), lambda i: (i, 0)),
        out_shape=jax.ShapeDtypeStruct((N, D), jnp.float32),
    )(x0, x1, q0, q1, wc, b2d)


def _fold_weights(w):
    a0 = w[:, 0::3].T
    a1 = w[:, 1::3].T
    a2 = w[:, 2::3].T
    return jnp.concatenate([a0 - a2, a1, 2.0 * a2], axis=0)


def kernel(x, edge_weight, W0, b0, W1, b1, W2, b2, edge_index):
    row = edge_index[0]
    col = edge_index[1]
    zeros = jnp.zeros((N, D), jnp.float32)
    h = x
    params = [(W0, b0, True), (W1, b1, True), (W2, b2, False)]
    for w, b, relu in params:
        wc = _fold_weights(w)
        p = _spmm_sc(h, row, col, edge_weight, zeros)
        x1 = _pair_add(p[0], p[1])
        q = _spmm_sc(x1, row, col, edge_weight, zeros)
        h = _layer_tc(h, x1, q[0], q[1], wc, b.reshape(1, D), relu)
    return h
